# Initial kernel scaffold; baseline (speedup 1.0000x reference)
import jax
import jax.numpy as jnp
from jax.experimental import pallas as pl


def kernel(features, edge_index, W1, b1, W2, b2, W3, b3, W4, b4, W5, b5, W6, b6, W7, b7, W8, b8):
    raise NotImplementedError("write your pallas kernel here")


# speedup vs baseline: 4.4596x; 4.4596x over previous
import functools

import jax
import jax.numpy as jnp
from jax import lax
from jax.experimental import pallas as pl
from jax.experimental.pallas import tpu as pltpu
from jax.experimental.pallas import tpu_sc as plsc

N = 100000
NPAD = 100352
SUB = NPAD // 16
E = 1600000
EPAD = 1638400
R = EPAD // 128
ROWS_W64 = R // 16
ROWS_W4 = R // 32
CH = 8
BN = 1024
G = NPAD // BN

_MESH = plsc.VectorSubcoreMesh(core_axis_name="c", subcore_axis_name="s")
_SC_PARAMS = pltpu.CompilerParams(use_tc_tiling_on_sc=False)
_f32 = jnp.float32


def _deg_body(src2, dst2, ones8, zer8, deg2, obuf, ibuf, deg_sh):
    c = lax.axis_index("c")
    s = lax.axis_index("s")
    pltpu.sync_copy(ones8, obuf)
    pltpu.sync_copy(zer8, deg_sh.at[pl.ds(s * SUB, SUB)])
    plsc.subcore_barrier()

    def scan_edges(idx2):
        def chunk(k, _):
            base = s * ROWS_W64 + k * CH
            pltpu.sync_copy(idx2.at[pl.ds(base, CH)], ibuf)
            for j in range(CH):
                pltpu.sync_copy(obuf, deg_sh.at[ibuf.at[j]], add=True)
            return _
        lax.fori_loop(0, ROWS_W64 // CH, chunk, None)

    pl.when(c == 0)(lambda: scan_edges(src2))
    pl.when(c == 1)(lambda: scan_edges(dst2))
    plsc.subcore_barrier()
    pltpu.sync_copy(deg_sh.at[pl.ds(s * SUB, SUB)],
                    deg2.at[c, pl.ds(s * SUB, SUB)])


_deg_call = pl.kernel(
    _deg_body,
    out_type=jax.ShapeDtypeStruct((2, NPAD, 8), _f32),
    mesh=_MESH,
    compiler_params=_SC_PARAMS,
    scratch_types=[
        pltpu.VMEM((128, 8), _f32),
        pltpu.VMEM((CH, 128), jnp.int32),
        pltpu.VMEM_SHARED((NPAD, 8), _f32),
    ],
)


def _prop_w4_body(table, src2, dst2, zer16, agg2,
                  sbuf, dbuf, rbuf, gsem, agg_sh):
    c = lax.axis_index("c")
    s = lax.axis_index("s")
    pltpu.sync_copy(zer16, agg_sh.at[pl.ds(s * SUB, SUB)])
    plsc.subcore_barrier()

    def chunk(k, _):
        base = c * (R // 2) + s * ROWS_W4 + k * CH
        pltpu.sync_copy(src2.at[pl.ds(base, CH)], sbuf)
        pltpu.sync_copy(dst2.at[pl.ds(base, CH)], dbuf)
        descs = [pltpu.async_copy(table.at[sbuf.at[j]], rbuf.at[j], gsem)
                 for j in range(CH)]
        for j in range(CH):
            descs[j].wait()
        for j in range(CH):
            pltpu.sync_copy(rbuf.at[j], agg_sh.at[dbuf.at[j]], add=True)
        return _

    lax.fori_loop(0, ROWS_W4 // CH, chunk, None)
    plsc.subcore_barrier()
    pltpu.sync_copy(agg_sh.at[pl.ds(s * SUB, SUB)],
                    agg2.at[c, pl.ds(s * SUB, SUB)])


_prop_w4_call = pl.kernel(
    _prop_w4_body,
    out_type=jax.ShapeDtypeStruct((2, NPAD, 16), _f32),
    mesh=_MESH,
    compiler_params=_SC_PARAMS,
    scratch_types=[
        pltpu.VMEM((CH, 128), jnp.int32),
        pltpu.VMEM((CH, 128), jnp.int32),
        pltpu.VMEM((CH, 128, 16), _f32),
        pltpu.SemaphoreType.DMA,
        pltpu.VMEM_SHARED((NPAD, 16), _f32),
    ],
)


def _prop_w64_body(t0, t1, t2, t3, src2, dst2, zer16, agg4,
                   sbuf, dbuf, rbuf, gsem, agg_sh):
    c = lax.axis_index("c")
    s = lax.axis_index("s")

    def zero_own_stripe():
        pltpu.sync_copy(zer16, agg_sh.at[pl.ds(s * SUB, SUB)])

    def accumulate(tq):
        def chunk(k, _):
            base = s * ROWS_W64 + k * CH
            pltpu.sync_copy(src2.at[pl.ds(base, CH)], sbuf)
            pltpu.sync_copy(dst2.at[pl.ds(base, CH)], dbuf)
            descs = [pltpu.async_copy(tq.at[sbuf.at[j]], rbuf.at[j], gsem)
                     for j in range(CH)]
            for j in range(CH):
                descs[j].wait()
            for j in range(CH):
                pltpu.sync_copy(rbuf.at[j], agg_sh.at[dbuf.at[j]], add=True)
            return _
        lax.fori_loop(0, ROWS_W64 // CH, chunk, None)

    def dump(q):
        pltpu.sync_copy(agg_sh.at[pl.ds(s * SUB, SUB)],
                        agg4.at[q, pl.ds(s * SUB, SUB)])

    def run_sc(qa, ta, qb, tb):
        zero_own_stripe()
        plsc.subcore_barrier()
        accumulate(ta)
        plsc.subcore_barrier()
        dump(qa)
        zero_own_stripe()
        plsc.subcore_barrier()
        accumulate(tb)
        plsc.subcore_barrier()
        dump(qb)

    pl.when(c == 0)(lambda: run_sc(0, t0, 2, t2))
    pl.when(c == 1)(lambda: run_sc(1, t1, 3, t3))


_prop_w64_call = pl.kernel(
    _prop_w64_body,
    out_type=jax.ShapeDtypeStruct((4, NPAD, 16), _f32),
    mesh=_MESH,
    compiler_params=_SC_PARAMS,
    scratch_types=[
        pltpu.VMEM((CH, 128), jnp.int32),
        pltpu.VMEM((CH, 128), jnp.int32),
        pltpu.VMEM((CH, 128, 16), _f32),
        pltpu.SemaphoreType.DMA,
        pltpu.VMEM_SHARED((NPAD, 16), _f32),
    ],
)


def _leaky(z):
    return jnp.where(z >= 0, z, 0.01 * z)


def _prep_body(deg_ref, f_ref, on_ref, in_ref, h0_ref):
    d = deg_ref[...]
    on = lax.rsqrt(jnp.maximum(d[0, :, 0:1], 1.0))
    inn = lax.rsqrt(jnp.maximum(d[1, :, 0:1], 1.0))
    on_ref[...] = on
    in_ref[...] = inn
    h0_ref[...] = jnp.pad(f_ref[...] * on, ((0, 0), (0, 12)))


_prep_call = pl.pallas_call(
    _prep_body,
    grid=(G,),
    in_specs=[
        pl.BlockSpec((2, BN, 8), lambda i: (0, i, 0)),
        pl.BlockSpec((BN, 4), lambda i: (i, 0)),
    ],
    out_specs=[
        pl.BlockSpec((BN, 1), lambda i: (i, 0)),
        pl.BlockSpec((BN, 1), lambda i: (i, 0)),
        pl.BlockSpec((BN, 16), lambda i: (i, 0)),
    ],
    out_shape=[
        jax.ShapeDtypeStruct((NPAD, 1), _f32),
        jax.ShapeDtypeStruct((NPAD, 1), _f32),
        jax.ShapeDtypeStruct((NPAD, 16), _f32),
    ],
)


def _l1_body(a_ref, in_ref, on_ref, w_ref, b_ref, o0, o1, o2, o3):
    a = a_ref[...]
    A = (a[0] + a[1])[:, :4] * in_ref[...]
    Z = jnp.dot(A, w_ref[...], preferred_element_type=_f32) + b_ref[...]
    H = _leaky(Z) * on_ref[...]
    for q, o in enumerate((o0, o1, o2, o3)):
        o[...] = H[:, q * 16:(q + 1) * 16]


_l1_call = pl.pallas_call(
    _l1_body,
    grid=(G,),
    in_specs=[
        pl.BlockSpec((2, BN, 16), lambda i: (0, i, 0)),
        pl.BlockSpec((BN, 1), lambda i: (i, 0)),
        pl.BlockSpec((BN, 1), lambda i: (i, 0)),
        pl.BlockSpec((4, 64), lambda i: (0, 0)),
        pl.BlockSpec((1, 64), lambda i: (0, 0)),
    ],
    out_specs=[pl.BlockSpec((BN, 16), lambda i: (i, 0))] * 4,
    out_shape=[jax.ShapeDtypeStruct((NPAD, 16), _f32)] * 4,
)


def _mid_body(with_w8, a_ref, in_ref, on_ref, w_ref, b_ref, *refs):
    if with_w8:
        w8_ref = refs[0]
        outs, h4_ref = refs[1:5], refs[5]
    else:
        outs = refs[0:4]
    a = a_ref[...]
    inn = in_ref[...]
    Z = b_ref[...]
    for q in range(4):
        Z = Z + jnp.dot(a[q] * inn, w_ref[q * 16:(q + 1) * 16, :],
                        preferred_element_type=_f32)
    H = _leaky(Z) * on_ref[...]
    for q, o in enumerate(outs):
        o[...] = H[:, q * 16:(q + 1) * 16]
    if with_w8:
        h4_ref[...] = jnp.dot(H, w8_ref[...], preferred_element_type=_f32)


def _make_mid(with_w8):
    in_specs = [
        pl.BlockSpec((4, BN, 16), lambda i: (0, i, 0)),
        pl.BlockSpec((BN, 1), lambda i: (i, 0)),
        pl.BlockSpec((BN, 1), lambda i: (i, 0)),
        pl.BlockSpec((64, 64), lambda i: (0, 0)),
        pl.BlockSpec((1, 64), lambda i: (0, 0)),
    ]
    out_specs = [pl.BlockSpec((BN, 16), lambda i: (i, 0))] * 4
    out_shape = [jax.ShapeDtypeStruct((NPAD, 16), _f32)] * 4
    if with_w8:
        in_specs.append(pl.BlockSpec((64, 16), lambda i: (0, 0)))
        out_specs = out_specs + [pl.BlockSpec((BN, 16), lambda i: (i, 0))]
        out_shape = out_shape + [jax.ShapeDtypeStruct((NPAD, 16), _f32)]
    return pl.pallas_call(
        functools.partial(_mid_body, with_w8),
        grid=(G,), in_specs=in_specs, out_specs=out_specs,
        out_shape=out_shape)


_mid_call = _make_mid(False)
_mid_w8_call = _make_mid(True)


def _final_body(a_ref, in_ref, b_ref, y_ref):
    a = a_ref[...]
    y_ref[...] = (a[0] + a[1])[:, :4] * in_ref[...] + b_ref[...]


_final_call = pl.pallas_call(
    _final_body,
    grid=(G,),
    in_specs=[
        pl.BlockSpec((2, BN, 16), lambda i: (0, i, 0)),
        pl.BlockSpec((BN, 1), lambda i: (i, 0)),
        pl.BlockSpec((1, 4), lambda i: (0, 0)),
    ],
    out_specs=pl.BlockSpec((BN, 4), lambda i: (i, 0)),
    out_shape=jax.ShapeDtypeStruct((NPAD, 4), _f32),
)


def _impl(features, edge_index, W1, b1, W2, b2, W3, b3, W4, b4,
          W5, b5, W6, b6, W7, b7, W8, b8):
    e32 = edge_index.astype(jnp.int32)
    pad = jnp.full((EPAD - E,), N, jnp.int32)
    src2 = jnp.concatenate([e32[0], pad]).reshape(R, 128)
    dst2 = jnp.concatenate([e32[1], pad]).reshape(R, 128)
    featp = jnp.pad(features, ((0, NPAD - N), (0, 0)))

    ones8 = jnp.zeros((128, 8), _f32).at[:, 0].set(1.0)
    zer8 = jnp.zeros((SUB, 8), _f32)
    zer16 = jnp.zeros((SUB, 16), _f32)
    W8p = jnp.pad(W8, ((0, 0), (0, 13)))
    b8p = jnp.pad(b8, (0, 1)).reshape(1, 4)

    deg2 = _deg_call(src2, dst2, ones8, zer8)
    onorm, inorm, h0 = _prep_call(deg2, featp)

    agg2 = _prop_w4_call(h0, src2, dst2, zer16)
    t = _l1_call(agg2, inorm, onorm, W1, b1.reshape(1, 64))

    for W, b in ((W2, b2), (W3, b3), (W4, b4), (W5, b5), (W6, b6)):
        agg4 = _prop_w64_call(*t, src2, dst2, zer16)
        t = _mid_call(agg4, inorm, onorm, W, b.reshape(1, 64))

    agg4 = _prop_w64_call(*t, src2, dst2, zer16)
    *_t7, h4 = _mid_w8_call(agg4, inorm, onorm, W7, b7.reshape(1, 64), W8p)

    agg2b = _prop_w4_call(h4, src2, dst2, zer16)
    y = _final_call(agg2b, inorm, b8p)
    return y[:N, :3]


def kernel(features, edge_index, W1, b1, W2, b2, W3, b3, W4, b4,
           W5, b5, W6, b6, W7, b7, W8, b8):
    return _impl(features, edge_index, W1, b1, W2, b2, W3, b3, W4, b4,
                 W5, b5, W6, b6, W7, b7, W8, b8)


# speedup vs baseline: 5.1955x; 1.1650x over previous
import functools

import jax
import jax.numpy as jnp
from jax import lax
from jax.experimental import pallas as pl
from jax.experimental.pallas import tpu as pltpu
from jax.experimental.pallas import tpu_sc as plsc

N = 100000
NPAD = 100352
SUB = NPAD // 16
E = 1600000
EPAD = 1638400
R = EPAD // 128
ROWS_W64 = R // 16
ROWS_W4 = R // 32
CH = 5
IB = 4 * CH
BN = 1024
G = NPAD // BN

_MESH = plsc.VectorSubcoreMesh(core_axis_name="c", subcore_axis_name="s")
_SC_PARAMS = pltpu.CompilerParams(use_tc_tiling_on_sc=False)
_f32 = jnp.float32


def _deg_body(src2, dst2, ones8, zer8, deg2, obuf, ibuf, ssem, deg_sh):
    c = lax.axis_index("c")
    s = lax.axis_index("s")
    pltpu.sync_copy(ones8, obuf)
    pltpu.sync_copy(zer8, deg_sh.at[pl.ds(s * SUB, SUB)])
    plsc.subcore_barrier()

    def scan_edges(idx2):
        def chunk(k, _):
            base = s * ROWS_W64 + k * IB
            pltpu.sync_copy(idx2.at[pl.ds(base, IB)], ibuf)
            descs = [pltpu.async_copy(obuf, deg_sh.at[ibuf.at[j]], ssem,
                                      add=True)
                     for j in range(IB)]
            for d in descs:
                d.wait()
            return _
        lax.fori_loop(0, ROWS_W64 // IB, chunk, None)

    pl.when(c == 0)(lambda: scan_edges(src2))
    pl.when(c == 1)(lambda: scan_edges(dst2))
    plsc.subcore_barrier()
    pltpu.sync_copy(deg_sh.at[pl.ds(s * SUB, SUB)],
                    deg2.at[c, pl.ds(s * SUB, SUB)])


_deg_call = pl.kernel(
    _deg_body,
    out_type=jax.ShapeDtypeStruct((2, NPAD, 8), _f32),
    mesh=_MESH,
    compiler_params=_SC_PARAMS,
    scratch_types=[
        pltpu.VMEM((128, 8), _f32),
        pltpu.VMEM((IB, 128), jnp.int32),
        pltpu.SemaphoreType.DMA,
        pltpu.VMEM_SHARED((NPAD, 8), _f32),
    ],
)


def _pipelined_block(tq, agg_sh, sbuf, dbuf, rbuf, gsem, ssem):
    def fire_g(slot, sub):
        return [pltpu.async_copy(tq.at[sbuf.at[sub * CH + j]],
                                 rbuf.at[slot].at[j], gsem)
                for j in range(CH)]

    def fire_s(slot, sub):
        return [pltpu.async_copy(rbuf.at[slot].at[j],
                                 agg_sh.at[dbuf.at[sub * CH + j]], ssem,
                                 add=True)
                for j in range(CH)]

    def drain(descs):
        for d in descs:
            d.wait()

    g0 = fire_g(0, 0)
    g1 = fire_g(1, 1)
    drain(g0)
    s0 = fire_s(0, 0)
    drain(g1)
    drain(s0)
    g2 = fire_g(0, 2)
    s1 = fire_s(1, 1)
    drain(g2)
    drain(s1)
    g3 = fire_g(1, 3)
    s2 = fire_s(0, 2)
    drain(g3)
    drain(s2)
    s3 = fire_s(1, 3)
    drain(s3)


def _prop_w4_body(table, src2, dst2, zer16, agg2,
                  sbuf, dbuf, rbuf, gsem, ssem, agg_sh):
    c = lax.axis_index("c")
    s = lax.axis_index("s")
    pltpu.sync_copy(zer16, agg_sh.at[pl.ds(s * SUB, SUB)])
    plsc.subcore_barrier()

    def block(k, _):
        base = c * (R // 2) + s * ROWS_W4 + k * IB
        pltpu.sync_copy(src2.at[pl.ds(base, IB)], sbuf)
        pltpu.sync_copy(dst2.at[pl.ds(base, IB)], dbuf)
        _pipelined_block(table, agg_sh, sbuf, dbuf, rbuf, gsem, ssem)
        return _

    lax.fori_loop(0, ROWS_W4 // IB, block, None)
    plsc.subcore_barrier()
    pltpu.sync_copy(agg_sh.at[pl.ds(s * SUB, SUB)],
                    agg2.at[c, pl.ds(s * SUB, SUB)])


_prop_w4_call = pl.kernel(
    _prop_w4_body,
    out_type=jax.ShapeDtypeStruct((2, NPAD, 16), _f32),
    mesh=_MESH,
    compiler_params=_SC_PARAMS,
    scratch_types=[
        pltpu.VMEM((IB, 128), jnp.int32),
        pltpu.VMEM((IB, 128), jnp.int32),
        pltpu.VMEM((2, CH, 128, 16), _f32),
        pltpu.SemaphoreType.DMA,
        pltpu.SemaphoreType.DMA,
        pltpu.VMEM_SHARED((NPAD, 16), _f32),
    ],
)


def _prop_w64_body(t0, t1, t2, t3, src2, dst2, zer16, agg4,
                   sbuf, dbuf, rbuf, gsem, ssem, agg_sh):
    c = lax.axis_index("c")
    s = lax.axis_index("s")

    def zero_own_stripe():
        pltpu.sync_copy(zer16, agg_sh.at[pl.ds(s * SUB, SUB)])

    def accumulate(tq):
        def block(k, _):
            base = s * ROWS_W64 + k * IB
            pltpu.sync_copy(src2.at[pl.ds(base, IB)], sbuf)
            pltpu.sync_copy(dst2.at[pl.ds(base, IB)], dbuf)
            _pipelined_block(tq, agg_sh, sbuf, dbuf, rbuf, gsem, ssem)
            return _
        lax.fori_loop(0, ROWS_W64 // IB, block, None)

    def dump(q):
        pltpu.sync_copy(agg_sh.at[pl.ds(s * SUB, SUB)],
                        agg4.at[q, pl.ds(s * SUB, SUB)])

    def run_sc(qa, ta, qb, tb):
        zero_own_stripe()
        plsc.subcore_barrier()
        accumulate(ta)
        plsc.subcore_barrier()
        dump(qa)
        zero_own_stripe()
        plsc.subcore_barrier()
        accumulate(tb)
        plsc.subcore_barrier()
        dump(qb)

    pl.when(c == 0)(lambda: run_sc(0, t0, 2, t2))
    pl.when(c == 1)(lambda: run_sc(1, t1, 3, t3))


_prop_w64_call = pl.kernel(
    _prop_w64_body,
    out_type=jax.ShapeDtypeStruct((4, NPAD, 16), _f32),
    mesh=_MESH,
    compiler_params=_SC_PARAMS,
    scratch_types=[
        pltpu.VMEM((IB, 128), jnp.int32),
        pltpu.VMEM((IB, 128), jnp.int32),
        pltpu.VMEM((2, CH, 128, 16), _f32),
        pltpu.SemaphoreType.DMA,
        pltpu.SemaphoreType.DMA,
        pltpu.VMEM_SHARED((NPAD, 16), _f32),
    ],
)


def _leaky(z):
    return jnp.where(z >= 0, z, 0.01 * z)


def _prep_body(deg_ref, f_ref, on_ref, in_ref, h0_ref):
    d = deg_ref[...]
    on = lax.rsqrt(jnp.maximum(d[0, :, 0:1], 1.0))
    inn = lax.rsqrt(jnp.maximum(d[1, :, 0:1], 1.0))
    on_ref[...] = on
    in_ref[...] = inn
    h0_ref[...] = jnp.pad(f_ref[...] * on, ((0, 0), (0, 12)))


_prep_call = pl.pallas_call(
    _prep_body,
    grid=(G,),
    in_specs=[
        pl.BlockSpec((2, BN, 8), lambda i: (0, i, 0)),
        pl.BlockSpec((BN, 4), lambda i: (i, 0)),
    ],
    out_specs=[
        pl.BlockSpec((BN, 1), lambda i: (i, 0)),
        pl.BlockSpec((BN, 1), lambda i: (i, 0)),
        pl.BlockSpec((BN, 16), lambda i: (i, 0)),
    ],
    out_shape=[
        jax.ShapeDtypeStruct((NPAD, 1), _f32),
        jax.ShapeDtypeStruct((NPAD, 1), _f32),
        jax.ShapeDtypeStruct((NPAD, 16), _f32),
    ],
)


def _l1_body(a_ref, in_ref, on_ref, w_ref, b_ref, o0, o1, o2, o3):
    a = a_ref[...]
    A = (a[0] + a[1])[:, :4] * in_ref[...]
    Z = jnp.dot(A, w_ref[...], preferred_element_type=_f32) + b_ref[...]
    H = _leaky(Z) * on_ref[...]
    for q, o in enumerate((o0, o1, o2, o3)):
        o[...] = H[:, q * 16:(q + 1) * 16]


_l1_call = pl.pallas_call(
    _l1_body,
    grid=(G,),
    in_specs=[
        pl.BlockSpec((2, BN, 16), lambda i: (0, i, 0)),
        pl.BlockSpec((BN, 1), lambda i: (i, 0)),
        pl.BlockSpec((BN, 1), lambda i: (i, 0)),
        pl.BlockSpec((4, 64), lambda i: (0, 0)),
        pl.BlockSpec((1, 64), lambda i: (0, 0)),
    ],
    out_specs=[pl.BlockSpec((BN, 16), lambda i: (i, 0))] * 4,
    out_shape=[jax.ShapeDtypeStruct((NPAD, 16), _f32)] * 4,
)


def _mid_body(with_w8, a_ref, in_ref, on_ref, w_ref, b_ref, *refs):
    if with_w8:
        w8_ref = refs[0]
        outs, h4_ref = refs[1:5], refs[5]
    else:
        outs = refs[0:4]
    a = a_ref[...]
    inn = in_ref[...]
    Z = b_ref[...]
    for q in range(4):
        Z = Z + jnp.dot(a[q] * inn, w_ref[q * 16:(q + 1) * 16, :],
                        preferred_element_type=_f32)
    H = _leaky(Z) * on_ref[...]
    for q, o in enumerate(outs):
        o[...] = H[:, q * 16:(q + 1) * 16]
    if with_w8:
        h4_ref[...] = jnp.dot(H, w8_ref[...], preferred_element_type=_f32)


def _make_mid(with_w8):
    in_specs = [
        pl.BlockSpec((4, BN, 16), lambda i: (0, i, 0)),
        pl.BlockSpec((BN, 1), lambda i: (i, 0)),
        pl.BlockSpec((BN, 1), lambda i: (i, 0)),
        pl.BlockSpec((64, 64), lambda i: (0, 0)),
        pl.BlockSpec((1, 64), lambda i: (0, 0)),
    ]
    out_specs = [pl.BlockSpec((BN, 16), lambda i: (i, 0))] * 4
    out_shape = [jax.ShapeDtypeStruct((NPAD, 16), _f32)] * 4
    if with_w8:
        in_specs.append(pl.BlockSpec((64, 16), lambda i: (0, 0)))
        out_specs = out_specs + [pl.BlockSpec((BN, 16), lambda i: (i, 0))]
        out_shape = out_shape + [jax.ShapeDtypeStruct((NPAD, 16), _f32)]
    return pl.pallas_call(
        functools.partial(_mid_body, with_w8),
        grid=(G,), in_specs=in_specs, out_specs=out_specs,
        out_shape=out_shape)


_mid_call = _make_mid(False)
_mid_w8_call = _make_mid(True)


def _final_body(a_ref, in_ref, b_ref, y_ref):
    a = a_ref[...]
    y_ref[...] = (a[0] + a[1])[:, :4] * in_ref[...] + b_ref[...]


_final_call = pl.pallas_call(
    _final_body,
    grid=(G,),
    in_specs=[
        pl.BlockSpec((2, BN, 16), lambda i: (0, i, 0)),
        pl.BlockSpec((BN, 1), lambda i: (i, 0)),
        pl.BlockSpec((1, 4), lambda i: (0, 0)),
    ],
    out_specs=pl.BlockSpec((BN, 4), lambda i: (i, 0)),
    out_shape=jax.ShapeDtypeStruct((NPAD, 4), _f32),
)


def _impl(features, edge_index, W1, b1, W2, b2, W3, b3, W4, b4,
          W5, b5, W6, b6, W7, b7, W8, b8):
    e32 = edge_index.astype(jnp.int32)
    pad = jnp.full((EPAD - E,), N, jnp.int32)
    src2 = jnp.concatenate([e32[0], pad]).reshape(R, 128)
    dst2 = jnp.concatenate([e32[1], pad]).reshape(R, 128)
    featp = jnp.pad(features, ((0, NPAD - N), (0, 0)))

    ones8 = jnp.zeros((128, 8), _f32).at[:, 0].set(1.0)
    zer8 = jnp.zeros((SUB, 8), _f32)
    zer16 = jnp.zeros((SUB, 16), _f32)
    W8p = jnp.pad(W8, ((0, 0), (0, 13)))
    b8p = jnp.pad(b8, (0, 1)).reshape(1, 4)

    deg2 = _deg_call(src2, dst2, ones8, zer8)
    onorm, inorm, h0 = _prep_call(deg2, featp)

    agg2 = _prop_w4_call(h0, src2, dst2, zer16)
    t = _l1_call(agg2, inorm, onorm, W1, b1.reshape(1, 64))

    for W, b in ((W2, b2), (W3, b3), (W4, b4), (W5, b5), (W6, b6)):
        agg4 = _prop_w64_call(*t, src2, dst2, zer16)
        t = _mid_call(agg4, inorm, onorm, W, b.reshape(1, 64))

    agg4 = _prop_w64_call(*t, src2, dst2, zer16)
    *_t7, h4 = _mid_w8_call(agg4, inorm, onorm, W7, b7.reshape(1, 64), W8p)

    agg2b = _prop_w4_call(h4, src2, dst2, zer16)
    y = _final_call(agg2b, inorm, b8p)
    return y[:N, :3]


def kernel(features, edge_index, W1, b1, W2, b2, W3, b3, W4, b4,
           W5, b5, W6, b6, W7, b7, W8, b8):
    return _impl(features, edge_index, W1, b1, W2, b2, W3, b3, W4, b4,
                 W5, b5, W6, b6, W7, b7, W8, b8)


# speedup vs baseline: 5.3338x; 1.0266x over previous
import functools

import jax
import jax.numpy as jnp
from jax import lax
from jax.experimental import pallas as pl
from jax.experimental.pallas import tpu as pltpu
from jax.experimental.pallas import tpu_sc as plsc

N = 100000
NPAD = 100352
SUB = NPAD // 16
E = 1600000
EPAD = 1638400
R = EPAD // 128
ROWS_W64 = R // 16
ROWS_W4 = R // 32
CH = 4
IB = 4 * CH
BN = 1024
G = NPAD // BN

_MESH = plsc.VectorSubcoreMesh(core_axis_name="c", subcore_axis_name="s")
_SC_PARAMS = pltpu.CompilerParams(use_tc_tiling_on_sc=False)
_f32 = jnp.float32


def _deg_body(src2, dst2, ones8, zer8, deg2, obuf, ibuf, ssem, deg_sh):
    c = lax.axis_index("c")
    s = lax.axis_index("s")
    pltpu.sync_copy(ones8, obuf)
    pltpu.sync_copy(zer8, deg_sh.at[pl.ds(s * SUB, SUB)])
    plsc.subcore_barrier()

    def scan_edges(idx2):
        def chunk(k, _):
            base = s * ROWS_W64 + k * IB
            pltpu.sync_copy(idx2.at[pl.ds(base, IB)], ibuf)
            descs = [pltpu.async_copy(obuf, deg_sh.at[ibuf.at[j]], ssem,
                                      add=True)
                     for j in range(IB)]
            for d in descs:
                d.wait()
            return _
        lax.fori_loop(0, ROWS_W64 // IB, chunk, None)

    pl.when(c == 0)(lambda: scan_edges(src2))
    pl.when(c == 1)(lambda: scan_edges(dst2))
    plsc.subcore_barrier()
    pltpu.sync_copy(deg_sh.at[pl.ds(s * SUB, SUB)],
                    deg2.at[c, pl.ds(s * SUB, SUB)])


_deg_call = pl.kernel(
    _deg_body,
    out_type=jax.ShapeDtypeStruct((2, NPAD, 8), _f32),
    mesh=_MESH,
    compiler_params=_SC_PARAMS,
    scratch_types=[
        pltpu.VMEM((128, 8), _f32),
        pltpu.VMEM((IB, 128), jnp.int32),
        pltpu.SemaphoreType.DMA,
        pltpu.VMEM_SHARED((NPAD, 8), _f32),
    ],
)


def _pipelined_block(tq, agg_sh, sbuf, dbuf, rbuf, gsem, ssem):
    def fire_g(slot, sub):
        return [pltpu.async_copy(tq.at[sbuf.at[sub * CH + j]],
                                 rbuf.at[slot].at[j], gsem)
                for j in range(CH)]

    def fire_s(slot, sub):
        return [pltpu.async_copy(rbuf.at[slot].at[j],
                                 agg_sh.at[dbuf.at[sub * CH + j]], ssem,
                                 add=True)
                for j in range(CH)]

    def drain(descs):
        for d in descs:
            d.wait()

    g0 = fire_g(0, 0)
    g1 = fire_g(1, 1)
    drain(g0)
    s0 = fire_s(0, 0)
    drain(g1)
    drain(s0)
    g2 = fire_g(0, 2)
    s1 = fire_s(1, 1)
    drain(g2)
    drain(s1)
    g3 = fire_g(1, 3)
    s2 = fire_s(0, 2)
    drain(g3)
    drain(s2)
    s3 = fire_s(1, 3)
    drain(s3)


def _prop_w4_body(table, src2, dst2, zer16, agg2,
                  sbuf, dbuf, rbuf, gsem, ssem, agg_sh):
    c = lax.axis_index("c")
    s = lax.axis_index("s")
    pltpu.sync_copy(zer16, agg_sh.at[pl.ds(s * SUB, SUB)])
    plsc.subcore_barrier()

    def block(k, _):
        base = c * (R // 2) + s * ROWS_W4 + k * IB
        pltpu.sync_copy(src2.at[pl.ds(base, IB)], sbuf)
        pltpu.sync_copy(dst2.at[pl.ds(base, IB)], dbuf)
        _pipelined_block(table, agg_sh, sbuf, dbuf, rbuf, gsem, ssem)
        return _

    lax.fori_loop(0, ROWS_W4 // IB, block, None)
    plsc.subcore_barrier()
    pltpu.sync_copy(agg_sh.at[pl.ds(s * SUB, SUB)],
                    agg2.at[c, pl.ds(s * SUB, SUB)])


_prop_w4_call = pl.kernel(
    _prop_w4_body,
    out_type=jax.ShapeDtypeStruct((2, NPAD, 16), _f32),
    mesh=_MESH,
    compiler_params=_SC_PARAMS,
    scratch_types=[
        pltpu.VMEM((IB, 128), jnp.int32),
        pltpu.VMEM((IB, 128), jnp.int32),
        pltpu.VMEM((2, CH, 128, 16), _f32),
        pltpu.SemaphoreType.DMA,
        pltpu.SemaphoreType.DMA,
        pltpu.VMEM_SHARED((NPAD, 16), _f32),
    ],
)


def _prop_w64_body(t0, t1, t2, t3, src2, dst2, zer16, agg4,
                   sbuf, dbuf, rbuf, gsem, ssem, isem, agg_sh):
    c = lax.axis_index("c")
    s = lax.axis_index("s")
    nb = ROWS_W64 // IB

    def zero_own_stripe():
        pltpu.sync_copy(zer16, agg_sh.at[pl.ds(s * SUB, SUB)])

    def accumulate(tq):
        def fire_idx(slot, blk):
            base = s * ROWS_W64 + blk * IB
            pltpu.async_copy(src2.at[pl.ds(base, IB)], sbuf.at[slot], isem)
            pltpu.async_copy(dst2.at[pl.ds(base, IB)], dbuf.at[slot], isem)

        def wait_idx(slot):
            pltpu.make_async_copy(src2.at[pl.ds(0, IB)], sbuf.at[slot],
                                  isem).wait()
            pltpu.make_async_copy(dst2.at[pl.ds(0, IB)], dbuf.at[slot],
                                  isem).wait()

        fire_idx(0, 0)
        wait_idx(0)

        def pair(m, _):
            fire_idx(1, 2 * m + 1)
            _pipelined_block(tq, agg_sh, sbuf.at[0], dbuf.at[0], rbuf,
                             gsem, ssem)
            wait_idx(1)
            fire_idx(0, lax.rem(2 * m + 2, nb))
            _pipelined_block(tq, agg_sh, sbuf.at[1], dbuf.at[1], rbuf,
                             gsem, ssem)
            wait_idx(0)
            return _
        lax.fori_loop(0, nb // 2, pair, None)

    def dump(q):
        pltpu.sync_copy(agg_sh.at[pl.ds(s * SUB, SUB)],
                        agg4.at[q, pl.ds(s * SUB, SUB)])

    def run_sc(qa, ta, qb, tb):
        zero_own_stripe()
        plsc.subcore_barrier()
        accumulate(ta)
        plsc.subcore_barrier()
        dump(qa)
        zero_own_stripe()
        plsc.subcore_barrier()
        accumulate(tb)
        plsc.subcore_barrier()
        dump(qb)

    pl.when(c == 0)(lambda: run_sc(0, t0, 2, t2))
    pl.when(c == 1)(lambda: run_sc(1, t1, 3, t3))


_prop_w64_call = pl.kernel(
    _prop_w64_body,
    out_type=jax.ShapeDtypeStruct((4, NPAD, 16), _f32),
    mesh=_MESH,
    compiler_params=_SC_PARAMS,
    scratch_types=[
        pltpu.VMEM((2, IB, 128), jnp.int32),
        pltpu.VMEM((2, IB, 128), jnp.int32),
        pltpu.VMEM((2, CH, 128, 16), _f32),
        pltpu.SemaphoreType.DMA,
        pltpu.SemaphoreType.DMA,
        pltpu.SemaphoreType.DMA,
        pltpu.VMEM_SHARED((NPAD, 16), _f32),
    ],
)


def _leaky(z):
    return jnp.where(z >= 0, z, 0.01 * z)


def _prep_body(deg_ref, f_ref, on_ref, in_ref, h0_ref):
    d = deg_ref[...]
    on = lax.rsqrt(jnp.maximum(d[0, :, 0:1], 1.0))
    inn = lax.rsqrt(jnp.maximum(d[1, :, 0:1], 1.0))
    on_ref[...] = on
    in_ref[...] = inn
    h0_ref[...] = jnp.pad(f_ref[...] * on, ((0, 0), (0, 12)))


_prep_call = pl.pallas_call(
    _prep_body,
    grid=(G,),
    in_specs=[
        pl.BlockSpec((2, BN, 8), lambda i: (0, i, 0)),
        pl.BlockSpec((BN, 4), lambda i: (i, 0)),
    ],
    out_specs=[
        pl.BlockSpec((BN, 1), lambda i: (i, 0)),
        pl.BlockSpec((BN, 1), lambda i: (i, 0)),
        pl.BlockSpec((BN, 16), lambda i: (i, 0)),
    ],
    out_shape=[
        jax.ShapeDtypeStruct((NPAD, 1), _f32),
        jax.ShapeDtypeStruct((NPAD, 1), _f32),
        jax.ShapeDtypeStruct((NPAD, 16), _f32),
    ],
)


def _l1_body(a_ref, in_ref, on_ref, w_ref, b_ref, o0, o1, o2, o3):
    a = a_ref[...]
    A = (a[0] + a[1])[:, :4] * in_ref[...]
    Z = jnp.dot(A, w_ref[...], preferred_element_type=_f32) + b_ref[...]
    H = _leaky(Z) * on_ref[...]
    for q, o in enumerate((o0, o1, o2, o3)):
        o[...] = H[:, q * 16:(q + 1) * 16]


_l1_call = pl.pallas_call(
    _l1_body,
    grid=(G,),
    in_specs=[
        pl.BlockSpec((2, BN, 16), lambda i: (0, i, 0)),
        pl.BlockSpec((BN, 1), lambda i: (i, 0)),
        pl.BlockSpec((BN, 1), lambda i: (i, 0)),
        pl.BlockSpec((4, 64), lambda i: (0, 0)),
        pl.BlockSpec((1, 64), lambda i: (0, 0)),
    ],
    out_specs=[pl.BlockSpec((BN, 16), lambda i: (i, 0))] * 4,
    out_shape=[jax.ShapeDtypeStruct((NPAD, 16), _f32)] * 4,
)


def _mid_body(with_w8, a_ref, in_ref, on_ref, w_ref, b_ref, *refs):
    if with_w8:
        w8_ref = refs[0]
        outs, h4_ref = refs[1:5], refs[5]
    else:
        outs = refs[0:4]
    a = a_ref[...]
    inn = in_ref[...]
    Z = b_ref[...]
    for q in range(4):
        Z = Z + jnp.dot(a[q] * inn, w_ref[q * 16:(q + 1) * 16, :],
                        preferred_element_type=_f32)
    H = _leaky(Z) * on_ref[...]
    for q, o in enumerate(outs):
        o[...] = H[:, q * 16:(q + 1) * 16]
    if with_w8:
        h4_ref[...] = jnp.dot(H, w8_ref[...], preferred_element_type=_f32)


def _make_mid(with_w8):
    in_specs = [
        pl.BlockSpec((4, BN, 16), lambda i: (0, i, 0)),
        pl.BlockSpec((BN, 1), lambda i: (i, 0)),
        pl.BlockSpec((BN, 1), lambda i: (i, 0)),
        pl.BlockSpec((64, 64), lambda i: (0, 0)),
        pl.BlockSpec((1, 64), lambda i: (0, 0)),
    ]
    out_specs = [pl.BlockSpec((BN, 16), lambda i: (i, 0))] * 4
    out_shape = [jax.ShapeDtypeStruct((NPAD, 16), _f32)] * 4
    if with_w8:
        in_specs.append(pl.BlockSpec((64, 16), lambda i: (0, 0)))
        out_specs = out_specs + [pl.BlockSpec((BN, 16), lambda i: (i, 0))]
        out_shape = out_shape + [jax.ShapeDtypeStruct((NPAD, 16), _f32)]
    return pl.pallas_call(
        functools.partial(_mid_body, with_w8),
        grid=(G,), in_specs=in_specs, out_specs=out_specs,
        out_shape=out_shape)


_mid_call = _make_mid(False)
_mid_w8_call = _make_mid(True)


def _final_body(a_ref, in_ref, b_ref, y_ref):
    a = a_ref[...]
    y_ref[...] = (a[0] + a[1])[:, :4] * in_ref[...] + b_ref[...]


_final_call = pl.pallas_call(
    _final_body,
    grid=(G,),
    in_specs=[
        pl.BlockSpec((2, BN, 16), lambda i: (0, i, 0)),
        pl.BlockSpec((BN, 1), lambda i: (i, 0)),
        pl.BlockSpec((1, 4), lambda i: (0, 0)),
    ],
    out_specs=pl.BlockSpec((BN, 4), lambda i: (i, 0)),
    out_shape=jax.ShapeDtypeStruct((NPAD, 4), _f32),
)


def _impl(features, edge_index, W1, b1, W2, b2, W3, b3, W4, b4,
          W5, b5, W6, b6, W7, b7, W8, b8):
    e32 = edge_index.astype(jnp.int32)
    pad = jnp.full((EPAD - E,), N, jnp.int32)
    src2 = jnp.concatenate([e32[0], pad]).reshape(R, 128)
    dst2 = jnp.concatenate([e32[1], pad]).reshape(R, 128)
    featp = jnp.pad(features, ((0, NPAD - N), (0, 0)))

    ones8 = jnp.zeros((128, 8), _f32).at[:, 0].set(1.0)
    zer8 = jnp.zeros((SUB, 8), _f32)
    zer16 = jnp.zeros((SUB, 16), _f32)
    W8p = jnp.pad(W8, ((0, 0), (0, 13)))
    b8p = jnp.pad(b8, (0, 1)).reshape(1, 4)

    deg2 = _deg_call(src2, dst2, ones8, zer8)
    onorm, inorm, h0 = _prep_call(deg2, featp)

    agg2 = _prop_w4_call(h0, src2, dst2, zer16)
    t = _l1_call(agg2, inorm, onorm, W1, b1.reshape(1, 64))

    for W, b in ((W2, b2), (W3, b3), (W4, b4), (W5, b5), (W6, b6)):
        agg4 = _prop_w64_call(*t, src2, dst2, zer16)
        t = _mid_call(agg4, inorm, onorm, W, b.reshape(1, 64))

    agg4 = _prop_w64_call(*t, src2, dst2, zer16)
    *_t7, h4 = _mid_w8_call(agg4, inorm, onorm, W7, b7.reshape(1, 64), W8p)

    agg2b = _prop_w4_call(h4, src2, dst2, zer16)
    y = _final_call(agg2b, inorm, b8p)
    return y[:N, :3]


def kernel(features, edge_index, W1, b1, W2, b2, W3, b3, W4, b4,
           W5, b5, W6, b6, W7, b7, W8, b8):
    return _impl(features, edge_index, W1, b1, W2, b2, W3, b3, W4, b4,
                 W5, b5, W6, b6, W7, b7, W8, b8)
